# baseline (speedup 1.0000x reference)
import functools

import jax
import jax.numpy as jnp
from jax import lax
from jax.experimental import pallas as pl
from jax.experimental.pallas import tpu as pltpu
from jax.experimental.pallas import tpu_sc as plsc

_N = 10000
_D = 256
_DH = 128
_E = 160000
_NT = 16
_GRP = 64
_NG = 160
_NR = 5
_EPT = _GRP * _NG
_EPAD = _NT * _EPT
_RPT = 624
_BM = 400


def _mm_split_body(a_ref, w_ref, b_ref, lo_ref, hi_ref):
    r = jnp.dot(a_ref[...], w_ref[...], preferred_element_type=jnp.float32)
    r = r + b_ref[...]
    lo_ref[...] = r[:, :_DH]
    hi_ref[...] = r[:, _DH:]


def _mm_split(a, w, b):
    n = a.shape[0]
    return pl.pallas_call(
        _mm_split_body,
        grid=(n // _BM,),
        in_specs=[
            pl.BlockSpec((_BM, _D), lambda i: (i, 0)),
            pl.BlockSpec((_D, _D), lambda i: (0, 0)),
            pl.BlockSpec((1, _D), lambda i: (0, 0)),
        ],
        out_specs=[
            pl.BlockSpec((_BM, _DH), lambda i: (i, 0)),
            pl.BlockSpec((_BM, _DH), lambda i: (i, 0)),
        ],
        out_shape=[
            jax.ShapeDtypeStruct((n, _DH), jnp.float32),
            jax.ShapeDtypeStruct((n, _DH), jnp.float32),
        ],
    )(a, w, b.reshape(1, _D))


def _mm_pair_body(lo_ref, hi_ref, w_ref, o_ref):
    o_ref[...] = (
        jnp.dot(lo_ref[...], w_ref[:_DH, :], preferred_element_type=jnp.float32)
        + jnp.dot(hi_ref[...], w_ref[_DH:, :], preferred_element_type=jnp.float32)
    )


def _mm_pair(lo, hi, w):
    n = lo.shape[0]
    return pl.pallas_call(
        _mm_pair_body,
        grid=(n // _BM,),
        in_specs=[
            pl.BlockSpec((_BM, _DH), lambda i: (i, 0)),
            pl.BlockSpec((_BM, _DH), lambda i: (i, 0)),
            pl.BlockSpec((_D, _D), lambda i: (0, 0)),
        ],
        out_specs=pl.BlockSpec((_BM, _D), lambda i: (i, 0)),
        out_shape=jax.ShapeDtypeStruct((n, _D), jnp.float32),
    )(lo, hi, w)


def _mm_pair_acc_body(lo_ref, hi_ref, w_ref, a_ref, o_ref):
    o_ref[...] = (
        a_ref[...]
        + jnp.dot(lo_ref[...], w_ref[:_DH, :], preferred_element_type=jnp.float32)
        + jnp.dot(hi_ref[...], w_ref[_DH:, :], preferred_element_type=jnp.float32)
    )


def _mm_pair_acc(lo, hi, w, acc):
    n = lo.shape[0]
    return pl.pallas_call(
        _mm_pair_acc_body,
        grid=(n // _BM,),
        in_specs=[
            pl.BlockSpec((_BM, _DH), lambda i: (i, 0)),
            pl.BlockSpec((_BM, _DH), lambda i: (i, 0)),
            pl.BlockSpec((_D, _D), lambda i: (0, 0)),
            pl.BlockSpec((_BM, _D), lambda i: (i, 0)),
        ],
        out_specs=pl.BlockSpec((_BM, _D), lambda i: (i, 0)),
        out_shape=jax.ShapeDtypeStruct((n, _D), jnp.float32),
        input_output_aliases={3: 0},
    )(lo, hi, w, acc)


def _mm_pair_split_body(lo_ref, hi_ref, w_ref, b_ref, olo_ref, ohi_ref):
    r = (
        jnp.dot(lo_ref[...], w_ref[:_DH, :], preferred_element_type=jnp.float32)
        + jnp.dot(hi_ref[...], w_ref[_DH:, :], preferred_element_type=jnp.float32)
        + b_ref[...]
    )
    olo_ref[...] = r[:, :_DH]
    ohi_ref[...] = r[:, _DH:]


def _mm_pair_split(lo, hi, w, b):
    n = lo.shape[0]
    return pl.pallas_call(
        _mm_pair_split_body,
        grid=(n // _BM,),
        in_specs=[
            pl.BlockSpec((_BM, _DH), lambda i: (i, 0)),
            pl.BlockSpec((_BM, _DH), lambda i: (i, 0)),
            pl.BlockSpec((_D, _D), lambda i: (0, 0)),
            pl.BlockSpec((1, _D), lambda i: (0, 0)),
        ],
        out_specs=[
            pl.BlockSpec((_BM, _DH), lambda i: (i, 0)),
            pl.BlockSpec((_BM, _DH), lambda i: (i, 0)),
        ],
        out_shape=[
            jax.ShapeDtypeStruct((n, _DH), jnp.float32),
            jax.ShapeDtypeStruct((n, _DH), jnp.float32),
        ],
    )(lo, hi, w, b.reshape(1, _D))


def _mm_round_body(lo_ref, hi_ref, wm_ref, wf_ref, b_ref, a_ref,
                   olo_ref, ohi_ref, oa_ref):
    lo = lo_ref[...]
    hi = hi_ref[...]
    h = (
        jnp.dot(lo, wm_ref[:_DH, :], preferred_element_type=jnp.float32)
        + jnp.dot(hi, wm_ref[_DH:, :], preferred_element_type=jnp.float32)
    )
    oa_ref[...] = a_ref[...] + h
    t = (
        jnp.dot(lo, wf_ref[:_DH, :], preferred_element_type=jnp.float32)
        + jnp.dot(hi, wf_ref[_DH:, :], preferred_element_type=jnp.float32)
        + b_ref[...]
    )
    olo_ref[...] = t[:, :_DH]
    ohi_ref[...] = t[:, _DH:]


def _mm_round(lo, hi, wm, wf, b, acc):
    n = lo.shape[0]
    return pl.pallas_call(
        _mm_round_body,
        grid=(n // _BM,),
        in_specs=[
            pl.BlockSpec((_BM, _DH), lambda i: (i, 0)),
            pl.BlockSpec((_BM, _DH), lambda i: (i, 0)),
            pl.BlockSpec((_D, _D), lambda i: (0, 0)),
            pl.BlockSpec((_D, _D), lambda i: (0, 0)),
            pl.BlockSpec((1, _D), lambda i: (0, 0)),
            pl.BlockSpec((_BM, _D), lambda i: (i, 0)),
        ],
        out_specs=[
            pl.BlockSpec((_BM, _DH), lambda i: (i, 0)),
            pl.BlockSpec((_BM, _DH), lambda i: (i, 0)),
            pl.BlockSpec((_BM, _D), lambda i: (i, 0)),
        ],
        out_shape=[
            jax.ShapeDtypeStruct((n, _DH), jnp.float32),
            jax.ShapeDtypeStruct((n, _DH), jnp.float32),
            jax.ShapeDtypeStruct((n, _D), jnp.float32),
        ],
        input_output_aliases={5: 2},
    )(lo, hi, wm, wf, b.reshape(1, _D), acc)


def _wmm_body(wm_ref, wl_ref, o_ref):
    o_ref[0] = jnp.dot(wm_ref[0], wl_ref[...],
                       preferred_element_type=jnp.float32)


def _wmm(w_meta, w_lin):
    return pl.pallas_call(
        _wmm_body,
        grid=(w_meta.shape[0],),
        in_specs=[
            pl.BlockSpec((1, _D, _D), lambda i: (i, 0, 0)),
            pl.BlockSpec((_D, _D), lambda i: (0, 0)),
        ],
        out_specs=pl.BlockSpec((1, _D, _D), lambda i: (i, 0, 0)),
        out_shape=jax.ShapeDtypeStruct(w_meta.shape, jnp.float32),
    )(w_meta, w_lin)


def _seg_body(t_lo, t_hi, src2, dst2, w2, zrows,
              out_lo, out_hi,
              srcb, dstb, wb, rows_v, acc_sh,
              si0, si1, si2, si3, si4,
              sg0, sg1, sg2, sg3, sg4,
              ss0, ss1, ss2, ss3, ss4):
    c = lax.axis_index("c")
    s = lax.axis_index("s")
    rbase = s * _RPT
    gbase = s * _NG

    sis = (si0, si1, si2, si3, si4)
    sgs = (sg0, sg1, sg2, sg3, sg4)
    sss = (ss0, ss1, ss2, ss3, ss4)

    def idx_start(g, r):
        pltpu.async_copy(src2.at[gbase + g], srcb.at[r], sis[r])
        pltpu.async_copy(dst2.at[gbase + g], dstb.at[r], sis[r])
        pltpu.async_copy(w2.at[gbase + g], wb.at[r], sis[r])

    def idx_wait(r):
        pltpu.make_async_copy(src2.at[0], srcb.at[r], sis[r]).wait()
        pltpu.make_async_copy(dst2.at[0], dstb.at[r], sis[r]).wait()
        pltpu.make_async_copy(w2.at[0], wb.at[r], sis[r]).wait()

    def gather_start(r):
        @pl.when(c == 0)
        def _():
            pltpu.async_copy(t_lo.at[srcb.at[r]], rows_v.at[r], sgs[r])

        @pl.when(c == 1)
        def _():
            pltpu.async_copy(t_hi.at[srcb.at[r]], rows_v.at[r], sgs[r])

    def gather_wait(r):
        pltpu.make_async_copy(t_lo.at[srcb.at[r]], rows_v.at[r],
                              sgs[r]).wait()

    def scale(r):
        def sub(jj, _):
            for h in range(2):
                w16 = wb[r, pl.ds(jj * 32 + h * 16, 16)]
                for j in range(16):
                    wsp = lax.gather(
                        w16, jnp.full((16, 1), j, jnp.int32),
                        lax.GatherDimensionNumbers(
                            offset_dims=(), collapsed_slice_dims=(0,),
                            start_index_map=(0,)),
                        (1,), mode=lax.GatherScatterMode.PROMISE_IN_BOUNDS)
                    row = jj * 32 + h * 16 + j
                    for k in range(_DH // 16):
                        sl = rows_v[r, row, k * 16:(k + 1) * 16]
                        rows_v[r, row, k * 16:(k + 1) * 16] = sl * wsp
            return 0
        lax.fori_loop(0, _GRP // 32, sub, 0)

    def scatter_start(r):
        pltpu.async_copy(rows_v.at[r], acc_sh.at[dstb.at[r]], sss[r],
                         add=True)

    def scatter_wait(r):
        pltpu.make_async_copy(rows_v.at[r], acc_sh.at[dstb.at[0]],
                              sss[r]).wait()

    idx_start(0, 0)
    idx_start(1, 1)
    idx_start(2, 2)
    idx_wait(0)
    gather_start(0)
    idx_wait(1)
    gather_start(1)

    pltpu.sync_copy(zrows.at[pl.ds(0, _RPT)], acc_sh.at[pl.ds(rbase, _RPT)])

    @pl.when(s == _NT - 1)
    def _():
        pltpu.sync_copy(zrows.at[pl.ds(0, 16)],
                        acc_sh.at[pl.ds(_NT * _RPT, 16)])

    plsc.subcore_barrier()

    _TLAST = _NG // _NR - 1

    def outer(t, _):
        for b in range(_NR):
            g = _NR * t + b
            rfree = (b + 3) % _NR
            rg = (b + 2) % _NR

            gather_wait(b)
            scale(b)

            if b < 2:
                @pl.when(t > 0)
                def _():
                    scatter_wait(rfree)
            else:
                scatter_wait(rfree)

            if b < 2:
                idx_start(g + 3, rfree)
            else:
                @pl.when(t < _TLAST)
                def _():
                    idx_start(g + 3, rfree)

            if b < 3:
                idx_wait(rg)
                gather_start(rg)
            else:
                @pl.when(t < _TLAST)
                def _():
                    idx_wait(rg)
                    gather_start(rg)

            scatter_start(b)
        return 0

    lax.fori_loop(0, _NG // _NR, outer, 0)
    scatter_wait(3)
    scatter_wait(4)

    plsc.subcore_barrier()

    tail = _NT * _RPT

    @pl.when(c == 0)
    def _():
        pltpu.sync_copy(acc_sh.at[pl.ds(rbase, _RPT)],
                        out_lo.at[pl.ds(rbase, _RPT)])

        @pl.when(s == _NT - 1)
        def _():
            pltpu.sync_copy(acc_sh.at[pl.ds(tail, 16)],
                            out_lo.at[pl.ds(tail, 16)])

    @pl.when(c == 1)
    def _():
        pltpu.sync_copy(acc_sh.at[pl.ds(rbase, _RPT)],
                        out_hi.at[pl.ds(rbase, _RPT)])

        @pl.when(s == _NT - 1)
        def _():
            pltpu.sync_copy(acc_sh.at[pl.ds(tail, 16)],
                            out_hi.at[pl.ds(tail, 16)])


_seg_call = pl.kernel(
    _seg_body,
    out_type=[
        jax.ShapeDtypeStruct((_N, _DH), jnp.float32),
        jax.ShapeDtypeStruct((_N, _DH), jnp.float32),
    ],
    mesh=plsc.VectorSubcoreMesh(core_axis_name="c", subcore_axis_name="s"),
    scratch_types=(
        [
            pltpu.VMEM((_NR, _GRP), jnp.int32),
            pltpu.VMEM((_NR, _GRP), jnp.int32),
            pltpu.VMEM((_NR, _GRP), jnp.float32),
            pltpu.VMEM((_NR, _GRP, _DH), jnp.float32),
            pltpu.VMEM_SHARED((_N, _DH), jnp.float32),
        ]
        + [pltpu.SemaphoreType.DMA] * (3 * _NR)
    ),
)


def kernel(x, edge_index, edge_weight, W_lin, b_lin, W_meta):
    src = edge_index[0]
    dst = edge_index[1]

    npad = _EPAD - _E
    pad_idx = (jnp.arange(npad, dtype=jnp.int32) * 13) % _N
    srcp = jnp.concatenate([src, pad_idx]).reshape(_NT * _NG, _GRP)
    dstp = jnp.concatenate([dst, pad_idx]).reshape(_NT * _NG, _GRP)
    wp = jnp.concatenate(
        [edge_weight, jnp.zeros((npad,), jnp.float32)]).reshape(
            _NT * _NG, _GRP)
    zrows = jnp.zeros((_RPT, _DH), jnp.float32)

    nmeta = W_meta.shape[0]
    Wmm = _wmm(W_meta, W_lin)
    t_lo, t_hi = _mm_split(x, W_lin, b_lin)
    out = None
    for i in range(nmeta):
        n_lo, n_hi = _seg_call(t_lo, t_hi, srcp, dstp, wp, zrows)
        if i < nmeta - 1:
            t_lo, t_hi = _mm_pair_split(n_lo, n_hi, Wmm[i], b_lin)
        if i == 0:
            out = _mm_pair(n_lo, n_hi, W_meta[0])
        else:
            out = _mm_pair_acc(n_lo, n_hi, W_meta[i], out)
    return out


# speedup vs baseline: 1.0802x; 1.0802x over previous
import jax
import jax.numpy as jnp
from jax import lax
from jax.experimental import pallas as pl
from jax.experimental.pallas import tpu as pltpu
from jax.experimental.pallas import tpu_sc as plsc

_N = 10000
_D = 256
_DH = 128
_E = 160000
_NT = 16
_GRP = 64
_NG = 160
_NR = 5
_EPT = _GRP * _NG
_EPAD = _NT * _EPT
_RPT = 624
_BM = 400


def _mm_split_body(a_ref, w_ref, b_ref, lo_ref, hi_ref):
    r = jnp.dot(a_ref[...], w_ref[...], preferred_element_type=jnp.float32)
    r = r + b_ref[...]
    lo_ref[...] = r[:, :_DH]
    hi_ref[...] = r[:, _DH:]


def _mm_split(a, w, b):
    n = a.shape[0]
    return pl.pallas_call(
        _mm_split_body,
        grid=(n // _BM,),
        in_specs=[
            pl.BlockSpec((_BM, _D), lambda i: (i, 0)),
            pl.BlockSpec((_D, _D), lambda i: (0, 0)),
            pl.BlockSpec((1, _D), lambda i: (0, 0)),
        ],
        out_specs=[
            pl.BlockSpec((_BM, _DH), lambda i: (i, 0)),
            pl.BlockSpec((_BM, _DH), lambda i: (i, 0)),
        ],
        out_shape=[
            jax.ShapeDtypeStruct((n, _DH), jnp.float32),
            jax.ShapeDtypeStruct((n, _DH), jnp.float32),
        ],
    )(a, w, b.reshape(1, _D))


def _mm_pair_body(lo_ref, hi_ref, w_ref, o_ref):
    o_ref[...] = (
        jnp.dot(lo_ref[...], w_ref[:_DH, :], preferred_element_type=jnp.float32)
        + jnp.dot(hi_ref[...], w_ref[_DH:, :], preferred_element_type=jnp.float32)
    )


def _mm_pair(lo, hi, w):
    n = lo.shape[0]
    return pl.pallas_call(
        _mm_pair_body,
        grid=(n // _BM,),
        in_specs=[
            pl.BlockSpec((_BM, _DH), lambda i: (i, 0)),
            pl.BlockSpec((_BM, _DH), lambda i: (i, 0)),
            pl.BlockSpec((_D, _D), lambda i: (0, 0)),
        ],
        out_specs=pl.BlockSpec((_BM, _D), lambda i: (i, 0)),
        out_shape=jax.ShapeDtypeStruct((n, _D), jnp.float32),
    )(lo, hi, w)


def _mm_pair_acc_body(lo_ref, hi_ref, w_ref, a_ref, o_ref):
    o_ref[...] = (
        a_ref[...]
        + jnp.dot(lo_ref[...], w_ref[:_DH, :], preferred_element_type=jnp.float32)
        + jnp.dot(hi_ref[...], w_ref[_DH:, :], preferred_element_type=jnp.float32)
    )


def _mm_pair_acc(lo, hi, w, acc):
    n = lo.shape[0]
    return pl.pallas_call(
        _mm_pair_acc_body,
        grid=(n // _BM,),
        in_specs=[
            pl.BlockSpec((_BM, _DH), lambda i: (i, 0)),
            pl.BlockSpec((_BM, _DH), lambda i: (i, 0)),
            pl.BlockSpec((_D, _D), lambda i: (0, 0)),
            pl.BlockSpec((_BM, _D), lambda i: (i, 0)),
        ],
        out_specs=pl.BlockSpec((_BM, _D), lambda i: (i, 0)),
        out_shape=jax.ShapeDtypeStruct((n, _D), jnp.float32),
        input_output_aliases={3: 0},
    )(lo, hi, w, acc)


def _mm_pair_split_body(lo_ref, hi_ref, w_ref, b_ref, olo_ref, ohi_ref):
    r = (
        jnp.dot(lo_ref[...], w_ref[:_DH, :], preferred_element_type=jnp.float32)
        + jnp.dot(hi_ref[...], w_ref[_DH:, :], preferred_element_type=jnp.float32)
        + b_ref[...]
    )
    olo_ref[...] = r[:, :_DH]
    ohi_ref[...] = r[:, _DH:]


def _mm_pair_split(lo, hi, w, b):
    n = lo.shape[0]
    return pl.pallas_call(
        _mm_pair_split_body,
        grid=(n // _BM,),
        in_specs=[
            pl.BlockSpec((_BM, _DH), lambda i: (i, 0)),
            pl.BlockSpec((_BM, _DH), lambda i: (i, 0)),
            pl.BlockSpec((_D, _D), lambda i: (0, 0)),
            pl.BlockSpec((1, _D), lambda i: (0, 0)),
        ],
        out_specs=[
            pl.BlockSpec((_BM, _DH), lambda i: (i, 0)),
            pl.BlockSpec((_BM, _DH), lambda i: (i, 0)),
        ],
        out_shape=[
            jax.ShapeDtypeStruct((n, _DH), jnp.float32),
            jax.ShapeDtypeStruct((n, _DH), jnp.float32),
        ],
    )(lo, hi, w, b.reshape(1, _D))


def _wmm_body(wm_ref, wl_ref, o_ref):
    o_ref[0] = jnp.dot(wm_ref[0], wl_ref[...],
                       preferred_element_type=jnp.float32)


def _wmm(w_meta, w_lin):
    return pl.pallas_call(
        _wmm_body,
        grid=(w_meta.shape[0],),
        in_specs=[
            pl.BlockSpec((1, _D, _D), lambda i: (i, 0, 0)),
            pl.BlockSpec((_D, _D), lambda i: (0, 0)),
        ],
        out_specs=pl.BlockSpec((1, _D, _D), lambda i: (i, 0, 0)),
        out_shape=jax.ShapeDtypeStruct(w_meta.shape, jnp.float32),
    )(w_meta, w_lin)


def _seg_body(t_lo, t_hi, src2, dst2, w2, zrows,
              out_lo, out_hi,
              srcb, dstb, wb, rows_v, acc_sh,
              si0, si1, si2, si3, si4,
              sg0, sg1, sg2, sg3, sg4,
              ss0, ss1, ss2, ss3, ss4):
    c = lax.axis_index("c")
    s = lax.axis_index("s")
    rbase = s * _RPT
    gbase = s * _NG

    sis = (si0, si1, si2, si3, si4)
    sgs = (sg0, sg1, sg2, sg3, sg4)
    sss = (ss0, ss1, ss2, ss3, ss4)

    def idx_start(g, r):
        pltpu.async_copy(src2.at[gbase + g], srcb.at[r], sis[r])
        pltpu.async_copy(dst2.at[gbase + g], dstb.at[r], sis[r])
        pltpu.async_copy(w2.at[gbase + g], wb.at[r], sis[r])

    def idx_wait(r):
        pltpu.make_async_copy(src2.at[0], srcb.at[r], sis[r]).wait()
        pltpu.make_async_copy(dst2.at[0], dstb.at[r], sis[r]).wait()
        pltpu.make_async_copy(w2.at[0], wb.at[r], sis[r]).wait()

    def gather_start(r):
        @pl.when(c == 0)
        def _():
            pltpu.async_copy(t_lo.at[srcb.at[r]], rows_v.at[r], sgs[r])

        @pl.when(c == 1)
        def _():
            pltpu.async_copy(t_hi.at[srcb.at[r]], rows_v.at[r], sgs[r])

    def gather_wait(r):
        pltpu.make_async_copy(t_lo.at[srcb.at[r]], rows_v.at[r],
                              sgs[r]).wait()

    def scale(r):
        def sub(jj, _):
            w16 = wb[r, pl.ds(jj * 16, 16)]
            for j in range(16):
                wsp = lax.gather(
                    w16, jnp.full((16, 1), j, jnp.int32),
                    lax.GatherDimensionNumbers(
                        offset_dims=(), collapsed_slice_dims=(0,),
                        start_index_map=(0,)),
                    (1,), mode=lax.GatherScatterMode.PROMISE_IN_BOUNDS)
                for k in range(_DH // 16):
                    sl = rows_v[r, jj * 16 + j, k * 16:(k + 1) * 16]
                    rows_v[r, jj * 16 + j, k * 16:(k + 1) * 16] = sl * wsp
            return 0
        lax.fori_loop(0, _GRP // 16, sub, 0)

    def scatter_start(r):
        pltpu.async_copy(rows_v.at[r], acc_sh.at[dstb.at[r]], sss[r],
                         add=True)

    def scatter_wait(r):
        pltpu.make_async_copy(rows_v.at[r], acc_sh.at[dstb.at[0]],
                              sss[r]).wait()

    idx_start(0, 0)
    idx_start(1, 1)
    idx_start(2, 2)
    idx_wait(0)
    gather_start(0)
    idx_wait(1)
    gather_start(1)

    pltpu.sync_copy(zrows.at[pl.ds(0, _RPT)], acc_sh.at[pl.ds(rbase, _RPT)])

    @pl.when(s == _NT - 1)
    def _():
        pltpu.sync_copy(zrows.at[pl.ds(0, 16)],
                        acc_sh.at[pl.ds(_NT * _RPT, 16)])

    plsc.subcore_barrier()

    _TLAST = _NG // _NR - 1

    def outer(t, _):
        for b in range(_NR):
            g = _NR * t + b
            rfree = (b + 3) % _NR
            rg = (b + 2) % _NR

            gather_wait(b)
            scale(b)

            if b < 2:
                @pl.when(t > 0)
                def _():
                    scatter_wait(rfree)
            else:
                scatter_wait(rfree)

            if b < 2:
                idx_start(g + 3, rfree)
            else:
                @pl.when(t < _TLAST)
                def _():
                    idx_start(g + 3, rfree)

            if b < 3:
                idx_wait(rg)
                gather_start(rg)
            else:
                @pl.when(t < _TLAST)
                def _():
                    idx_wait(rg)
                    gather_start(rg)

            scatter_start(b)
        return 0

    lax.fori_loop(0, _NG // _NR, outer, 0)
    scatter_wait(3)
    scatter_wait(4)

    plsc.subcore_barrier()

    tail = _NT * _RPT

    @pl.when(c == 0)
    def _():
        pltpu.sync_copy(acc_sh.at[pl.ds(rbase, _RPT)],
                        out_lo.at[pl.ds(rbase, _RPT)])

        @pl.when(s == _NT - 1)
        def _():
            pltpu.sync_copy(acc_sh.at[pl.ds(tail, 16)],
                            out_lo.at[pl.ds(tail, 16)])

    @pl.when(c == 1)
    def _():
        pltpu.sync_copy(acc_sh.at[pl.ds(rbase, _RPT)],
                        out_hi.at[pl.ds(rbase, _RPT)])

        @pl.when(s == _NT - 1)
        def _():
            pltpu.sync_copy(acc_sh.at[pl.ds(tail, 16)],
                            out_hi.at[pl.ds(tail, 16)])


_seg_call = pl.kernel(
    _seg_body,
    out_type=[
        jax.ShapeDtypeStruct((_N, _DH), jnp.float32),
        jax.ShapeDtypeStruct((_N, _DH), jnp.float32),
    ],
    mesh=plsc.VectorSubcoreMesh(core_axis_name="c", subcore_axis_name="s"),
    scratch_types=(
        [
            pltpu.VMEM((_NR, _GRP), jnp.int32),
            pltpu.VMEM((_NR, _GRP), jnp.int32),
            pltpu.VMEM((_NR, _GRP), jnp.float32),
            pltpu.VMEM((_NR, _GRP, _DH), jnp.float32),
            pltpu.VMEM_SHARED((_N, _DH), jnp.float32),
        ]
        + [pltpu.SemaphoreType.DMA] * (3 * _NR)
    ),
)


def kernel(x, edge_index, edge_weight, W_lin, b_lin, W_meta):
    src = edge_index[0]
    dst = edge_index[1]

    npad = _EPAD - _E
    pad_idx = (jnp.arange(npad, dtype=jnp.int32) * 13) % _N
    srcp = jnp.concatenate([src, pad_idx]).reshape(_NT * _NG, _GRP)
    dstp = jnp.concatenate([dst, pad_idx]).reshape(_NT * _NG, _GRP)
    wp = jnp.concatenate(
        [edge_weight, jnp.zeros((npad,), jnp.float32)]).reshape(
            _NT * _NG, _GRP)
    zrows = jnp.zeros((_RPT, _DH), jnp.float32)

    nmeta = W_meta.shape[0]
    Wmm = _wmm(W_meta, W_lin)
    t_lo, t_hi = _mm_split(x, W_lin, b_lin)
    out = None
    for i in range(nmeta):
        n_lo, n_hi = _seg_call(t_lo, t_hi, srcp, dstp, wp, zrows)
        if i < nmeta - 1:
            t_lo, t_hi = _mm_pair_split(n_lo, n_hi, Wmm[i], b_lin)
        if i == 0:
            out = _mm_pair(n_lo, n_hi, W_meta[0])
        else:
            out = _mm_pair_acc(n_lo, n_hi, W_meta[i], out)
    return out
